# Initial kernel scaffold; baseline (speedup 1.0000x reference)
import jax
import jax.numpy as jnp
from jax.experimental import pallas as pl


def kernel(inputs, embed):
    raise NotImplementedError("write your pallas kernel here")


# speedup vs baseline: 1.8984x; 1.8984x over previous
import functools

import jax
import jax.numpy as jnp
from jax import lax
from jax.experimental import pallas as pl
from jax.experimental.pallas import tpu as pltpu

DIM = 32
N_EMBED = 1024
BT = 1024


def _vq_block(x_ref, w_ref, q_ref, diff_ref, ind_ref):
    x = x_ref[...]
    w = w_ref[...]
    x2 = jnp.sum(x * x, axis=1, keepdims=True)
    e2 = jnp.sum(w * w, axis=0, keepdims=True)
    xw = jnp.dot(x, w, preferred_element_type=jnp.float32)
    dist = x2 - 2.0 * xw + e2
    ind = jnp.argmax(-dist, axis=1).astype(jnp.int32)
    onehot = (lax.broadcasted_iota(jnp.int32, (BT, N_EMBED), 1)
              == ind[:, None]).astype(jnp.float32)
    q = lax.dot_general(onehot, w, (((1,), (1,)), ((), ())),
                        preferred_element_type=jnp.float32)
    q_ref[...] = x + (q - x)
    diff_ref[...] = (q - x) ** 2
    ind_ref[...] = ind


def kernel(inputs, embed):
    n_tokens = inputs.shape[0]
    grid = (n_tokens // BT,)
    q, diff, ind = pl.pallas_call(
        _vq_block,
        grid=grid,
        in_specs=[
            pl.BlockSpec((BT, DIM), lambda i: (i, 0)),
            pl.BlockSpec((DIM, N_EMBED), lambda i: (0, 0)),
        ],
        out_specs=[
            pl.BlockSpec((BT, DIM), lambda i: (i, 0)),
            pl.BlockSpec((BT, DIM), lambda i: (i, 0)),
            pl.BlockSpec((BT,), lambda i: (i,)),
        ],
        out_shape=[
            jax.ShapeDtypeStruct((n_tokens, DIM), jnp.float32),
            jax.ShapeDtypeStruct((n_tokens, DIM), jnp.float32),
            jax.ShapeDtypeStruct((n_tokens,), jnp.int32),
        ],
    )(inputs, embed)
    return (q, diff.reshape(n_tokens, DIM, 1), ind.reshape(n_tokens, 1))


# speedup vs baseline: 2.0959x; 1.1040x over previous
import functools

import jax
import jax.numpy as jnp
from jax import lax
from jax.experimental import pallas as pl
from jax.experimental.pallas import tpu as pltpu

DIM = 32
N_EMBED = 1024
BT = 1024


def _vq_block(x_ref, w_ref, q_ref, diff_ref, ind_ref):
    x = x_ref[...]
    w = w_ref[...]
    e2 = jnp.sum(w * w, axis=0, keepdims=True)
    s = jnp.dot(x, w, preferred_element_type=jnp.float32) - 0.5 * e2
    ind = jnp.argmax(s, axis=1).astype(jnp.int32)
    onehot = (lax.broadcasted_iota(jnp.int32, (BT, N_EMBED), 1)
              == ind[:, None]).astype(jnp.float32)
    q = lax.dot_general(onehot, w, (((1,), (1,)), ((), ())),
                        preferred_element_type=jnp.float32)
    q_ref[...] = x + (q - x)
    diff_ref[...] = (q - x) ** 2
    ind_ref[...] = ind


def kernel(inputs, embed):
    n_tokens = inputs.shape[0]
    grid = (n_tokens // BT,)
    q, diff, ind = pl.pallas_call(
        _vq_block,
        grid=grid,
        in_specs=[
            pl.BlockSpec((BT, DIM), lambda i: (i, 0)),
            pl.BlockSpec((DIM, N_EMBED), lambda i: (0, 0)),
        ],
        out_specs=[
            pl.BlockSpec((BT, DIM), lambda i: (i, 0)),
            pl.BlockSpec((BT, DIM), lambda i: (i, 0)),
            pl.BlockSpec((BT,), lambda i: (i,)),
        ],
        out_shape=[
            jax.ShapeDtypeStruct((n_tokens, DIM), jnp.float32),
            jax.ShapeDtypeStruct((n_tokens, DIM), jnp.float32),
            jax.ShapeDtypeStruct((n_tokens,), jnp.int32),
        ],
    )(inputs, embed)
    return (q, diff.reshape(n_tokens, DIM, 1), ind.reshape(n_tokens, 1))


# speedup vs baseline: 2.8887x; 1.3783x over previous
import functools

import jax
import jax.numpy as jnp
from jax import lax
from jax.experimental import pallas as pl
from jax.experimental.pallas import tpu as pltpu

DIM = 32
N_EMBED = 1024
BT = 1024


def _vq_block(x_ref, w_ref, q_ref, diff_ref, ind_ref):
    x = x_ref[...]
    w = w_ref[...]
    e2 = jnp.sum(w * w, axis=0)
    sT = lax.dot_general(w, x, (((0,), (1,)), ((), ())),
                         preferred_element_type=jnp.float32)
    sT = sT - 0.5 * e2[:, None]
    ind = jnp.argmax(sT, axis=0).astype(jnp.int32)
    onehot = (lax.broadcasted_iota(jnp.int32, (N_EMBED, BT), 0)
              == ind[None, :]).astype(jnp.float32)
    qT = lax.dot_general(w, onehot, (((1,), (0,)), ((), ())),
                         preferred_element_type=jnp.float32)
    q = qT.T
    q_ref[...] = x + (q - x)
    diff_ref[...] = (q - x) ** 2
    ind_ref[...] = ind


def kernel(inputs, embed):
    n_tokens = inputs.shape[0]
    grid = (n_tokens // BT,)
    q, diff, ind = pl.pallas_call(
        _vq_block,
        grid=grid,
        in_specs=[
            pl.BlockSpec((BT, DIM), lambda i: (i, 0)),
            pl.BlockSpec((DIM, N_EMBED), lambda i: (0, 0)),
        ],
        out_specs=[
            pl.BlockSpec((BT, DIM), lambda i: (i, 0)),
            pl.BlockSpec((BT, DIM), lambda i: (i, 0)),
            pl.BlockSpec((BT,), lambda i: (i,)),
        ],
        out_shape=[
            jax.ShapeDtypeStruct((n_tokens, DIM), jnp.float32),
            jax.ShapeDtypeStruct((n_tokens, DIM), jnp.float32),
            jax.ShapeDtypeStruct((n_tokens,), jnp.int32),
        ],
    )(inputs, embed)
    return (q, diff.reshape(n_tokens, DIM, 1), ind.reshape(n_tokens, 1))


# speedup vs baseline: 3.2774x; 1.1346x over previous
import functools

import jax
import jax.numpy as jnp
from jax import lax
from jax.experimental import pallas as pl
from jax.experimental.pallas import tpu as pltpu

DIM = 32
N_EMBED = 1024
BT = 2048


def _vq_block(x_ref, w_ref, q_ref, diff_ref, ind_ref):
    x = x_ref[...]
    w = w_ref[...]
    e2 = jnp.sum(w * w, axis=0)
    sT = lax.dot_general(w, x, (((0,), (1,)), ((), ())),
                         preferred_element_type=jnp.float32)
    sT = sT - 0.5 * e2[:, None]
    ind = jnp.argmax(sT, axis=0).astype(jnp.int32)
    onehot = (lax.broadcasted_iota(jnp.int32, (N_EMBED, BT), 0)
              == ind[None, :]).astype(jnp.float32)
    qT = lax.dot_general(w, onehot, (((1,), (0,)), ((), ())),
                         preferred_element_type=jnp.float32)
    q = qT.T
    q_ref[...] = x + (q - x)
    diff_ref[...] = (q - x) ** 2
    ind_ref[...] = ind


def kernel(inputs, embed):
    n_tokens = inputs.shape[0]
    grid = (n_tokens // BT,)
    q, diff, ind = pl.pallas_call(
        _vq_block,
        grid=grid,
        in_specs=[
            pl.BlockSpec((BT, DIM), lambda i: (i, 0)),
            pl.BlockSpec((DIM, N_EMBED), lambda i: (0, 0)),
        ],
        out_specs=[
            pl.BlockSpec((BT, DIM), lambda i: (i, 0)),
            pl.BlockSpec((BT, DIM), lambda i: (i, 0)),
            pl.BlockSpec((BT,), lambda i: (i,)),
        ],
        out_shape=[
            jax.ShapeDtypeStruct((n_tokens, DIM), jnp.float32),
            jax.ShapeDtypeStruct((n_tokens, DIM), jnp.float32),
            jax.ShapeDtypeStruct((n_tokens,), jnp.int32),
        ],
    )(inputs, embed)
    return (q, diff.reshape(n_tokens, DIM, 1), ind.reshape(n_tokens, 1))


# speedup vs baseline: 3.4138x; 1.0416x over previous
import functools

import jax
import jax.numpy as jnp
from jax import lax
from jax.experimental import pallas as pl
from jax.experimental.pallas import tpu as pltpu

DIM = 32
N_EMBED = 1024
BT = 4096


def _vq_block(x_ref, w_ref, q_ref, diff_ref, ind_ref):
    x = x_ref[...]
    w = w_ref[...]
    e2 = jnp.sum(w * w, axis=0)
    sT = lax.dot_general(w, x, (((0,), (1,)), ((), ())),
                         preferred_element_type=jnp.float32)
    sT = sT - 0.5 * e2[:, None]
    ind = jnp.argmax(sT, axis=0).astype(jnp.int32)
    onehot = (lax.broadcasted_iota(jnp.int32, (N_EMBED, BT), 0)
              == ind[None, :]).astype(jnp.float32)
    qT = lax.dot_general(w, onehot, (((1,), (0,)), ((), ())),
                         preferred_element_type=jnp.float32)
    q = qT.T
    q_ref[...] = x + (q - x)
    diff_ref[...] = (q - x) ** 2
    ind_ref[...] = ind


def kernel(inputs, embed):
    n_tokens = inputs.shape[0]
    grid = (n_tokens // BT,)
    q, diff, ind = pl.pallas_call(
        _vq_block,
        grid=grid,
        in_specs=[
            pl.BlockSpec((BT, DIM), lambda i: (i, 0)),
            pl.BlockSpec((DIM, N_EMBED), lambda i: (0, 0)),
        ],
        out_specs=[
            pl.BlockSpec((BT, DIM), lambda i: (i, 0)),
            pl.BlockSpec((BT, DIM), lambda i: (i, 0)),
            pl.BlockSpec((BT,), lambda i: (i,)),
        ],
        out_shape=[
            jax.ShapeDtypeStruct((n_tokens, DIM), jnp.float32),
            jax.ShapeDtypeStruct((n_tokens, DIM), jnp.float32),
            jax.ShapeDtypeStruct((n_tokens,), jnp.int32),
        ],
    )(inputs, embed)
    return (q, diff.reshape(n_tokens, DIM, 1), ind.reshape(n_tokens, 1))


# speedup vs baseline: 3.4637x; 1.0146x over previous
import functools

import jax
import jax.numpy as jnp
from jax import lax
from jax.experimental import pallas as pl
from jax.experimental.pallas import tpu as pltpu

DIM = 32
N_EMBED = 1024
BT = 8192


def _vq_block(x_ref, w_ref, q_ref, diff_ref, ind_ref):
    x = x_ref[...]
    w = w_ref[...]
    e2 = jnp.sum(w * w, axis=0)
    sT = lax.dot_general(w, x, (((0,), (1,)), ((), ())),
                         preferred_element_type=jnp.float32)
    sT = sT - 0.5 * e2[:, None]
    ind = jnp.argmax(sT, axis=0).astype(jnp.int32)
    onehot = (lax.broadcasted_iota(jnp.int32, (N_EMBED, BT), 0)
              == ind[None, :]).astype(jnp.float32)
    qT = lax.dot_general(w, onehot, (((1,), (0,)), ((), ())),
                         preferred_element_type=jnp.float32)
    q = qT.T
    q_ref[...] = x + (q - x)
    diff_ref[...] = (q - x) ** 2
    ind_ref[...] = ind


def kernel(inputs, embed):
    n_tokens = inputs.shape[0]
    grid = (n_tokens // BT,)
    q, diff, ind = pl.pallas_call(
        _vq_block,
        grid=grid,
        in_specs=[
            pl.BlockSpec((BT, DIM), lambda i: (i, 0)),
            pl.BlockSpec((DIM, N_EMBED), lambda i: (0, 0)),
        ],
        out_specs=[
            pl.BlockSpec((BT, DIM), lambda i: (i, 0)),
            pl.BlockSpec((BT, DIM), lambda i: (i, 0)),
            pl.BlockSpec((BT,), lambda i: (i,)),
        ],
        out_shape=[
            jax.ShapeDtypeStruct((n_tokens, DIM), jnp.float32),
            jax.ShapeDtypeStruct((n_tokens, DIM), jnp.float32),
            jax.ShapeDtypeStruct((n_tokens,), jnp.int32),
        ],
    )(inputs, embed)
    return (q, diff.reshape(n_tokens, DIM, 1), ind.reshape(n_tokens, 1))
